# baseline (speedup 1.0000x reference)
import functools

import jax
import jax.numpy as jnp
from jax import lax
from jax.experimental import pallas as pl
from jax.experimental.pallas import tpu as pltpu
from jax.experimental.pallas import tpu_sc as plsc

EMBED = 64
NC = 2
NS = 16
NW = NC * NS
CHUNK = 128
LANES = 16


@functools.lru_cache(maxsize=None)
def _make_kernel(n_rows: int):
    n_per_w = n_rows // NW
    n_chunks = n_per_w // CHUNK

    mesh = plsc.VectorSubcoreMesh(core_axis_name="c", subcore_axis_name="s")

    @functools.partial(
        pl.kernel,
        mesh=mesh,
        out_type=jax.ShapeDtypeStruct((n_rows, EMBED), jnp.float32),
        scratch_types=[
            pltpu.VMEM((CHUNK,), jnp.int32),
            pltpu.VMEM((CHUNK,), jnp.int32),
            pltpu.VMEM((CHUNK, 2 * EMBED), jnp.float32),
            pltpu.VMEM((CHUNK, EMBED), jnp.float32),
            pltpu.SemaphoreType.DMA,
        ],
        compiler_params=pltpu.CompilerParams(needs_layout_passes=False),
    )
    def emb_kernel(idx_hbm, wp_hbm, out_hbm, xbuf, pbuf, rows, outb, gsem):
        wid = lax.axis_index("s") * NC + lax.axis_index("c")
        base = wid * n_per_w
        iota = lax.iota(jnp.int32, LANES)

        def body(j, carry):
            pltpu.sync_copy(idx_hbm.at[pl.ds(base + j * CHUNK, CHUNK)], xbuf)
            for g in range(CHUNK // LANES):
                xr = xbuf[pl.ds(g * LANES, LANES)]
                pbuf[pl.ds(g * LANES, LANES)] = lax.shift_right_logical(xr, 1)
            pltpu.async_copy(wp_hbm.at[pbuf], rows, gsem).wait()
            for g in range(CHUNK // LANES):
                xr = xbuf[pl.ds(g * LANES, LANES)]
                h64 = lax.shift_left(jnp.bitwise_and(xr, 1), 6)
                ri = iota + (g * LANES)
                for d in range(EMBED):
                    v = plsc.load_gather(rows, [ri, h64 + d])
                    plsc.store_scatter(
                        outb, [ri, jnp.full((LANES,), d, jnp.int32)], v
                    )
            pltpu.sync_copy(
                outb, out_hbm.at[pl.ds(base + j * CHUNK, CHUNK)]
            )
            return carry

        lax.fori_loop(0, n_chunks, body, 0)

    return emb_kernel


def kernel(x, weight):
    b, s = x.shape
    n_rows = b * s
    idx = x.reshape(n_rows).astype(jnp.int32)
    wp = weight.reshape(weight.shape[0] // 2, 2 * EMBED)
    out = _make_kernel(n_rows)(idx, wp)
    return out.reshape(b, s, EMBED)


# speedup vs baseline: 1.1010x; 1.1010x over previous
import functools

import jax
import jax.numpy as jnp
from jax import lax
from jax.experimental import pallas as pl
from jax.experimental.pallas import tpu as pltpu
from jax.experimental.pallas import tpu_sc as plsc

EMBED = 64
NC = 2
NS = 16
NW = NC * NS
CHUNK = 128
LANES = 16
GROUPS = CHUNK // LANES


@functools.lru_cache(maxsize=None)
def _make_kernel(n_rows: int):
    n_per_w = n_rows // NW
    n_chunks = n_per_w // CHUNK

    mesh = plsc.VectorSubcoreMesh(core_axis_name="c", subcore_axis_name="s")

    @functools.partial(
        pl.kernel,
        mesh=mesh,
        out_type=jax.ShapeDtypeStruct((n_rows, EMBED), jnp.float32),
        scratch_types=[
            pltpu.VMEM((n_chunks, CHUNK), jnp.int32),
            pltpu.VMEM((n_chunks, CHUNK), jnp.int32),
            pltpu.VMEM((CHUNK, 2 * EMBED), jnp.float32),
            pltpu.VMEM((CHUNK, 2 * EMBED), jnp.float32),
            pltpu.VMEM((CHUNK, EMBED), jnp.float32),
            pltpu.VMEM((CHUNK, EMBED), jnp.float32),
            pltpu.SemaphoreType.DMA,
            pltpu.SemaphoreType.DMA,
        ],
        compiler_params=pltpu.CompilerParams(needs_layout_passes=False),
    )
    def emb_kernel(p_hbm, h_hbm, wp_hbm, out_hbm,
                   pv, hv, rows_a, rows_b, out_a, out_b, gsem, osem):
        wid = lax.axis_index("s") * NC + lax.axis_index("c")
        cbase = wid * n_chunks
        base = wid * n_per_w
        iota = lax.iota(jnp.int32, LANES)

        pltpu.sync_copy(p_hbm.at[pl.ds(cbase, n_chunks)], pv)
        pltpu.sync_copy(h_hbm.at[pl.ds(cbase, n_chunks)], hv)

        def start_gather(c, rbuf):
            return pltpu.async_copy(wp_hbm.at[pv.at[c]], rbuf, gsem)

        def wait_gather(rbuf):
            pltpu.make_async_copy(wp_hbm.at[pv.at[0]], rbuf, gsem).wait()

        def wait_out(obuf):
            pltpu.make_async_copy(obuf, out_hbm.at[pl.ds(base, CHUNK)],
                                  osem).wait()

        def compact(c, rbuf, obuf):
            for g in range(GROUPS):
                ri = iota + (g * LANES)
                h64 = hv[c, pl.ds(g * LANES, LANES)]
                for d in range(EMBED):
                    v = plsc.load_gather(rbuf, [ri, h64 + d])
                    plsc.store_scatter(
                        obuf, [ri, jnp.full((LANES,), d, jnp.int32)], v
                    )

        def emit_out(c, obuf):
            pltpu.async_copy(obuf, out_hbm.at[pl.ds(base + c * CHUNK, CHUNK)],
                             osem)

        start_gather(0, rows_a)

        def body(j, carry):
            c0 = 2 * j
            c1 = c0 + 1

            wait_gather(rows_a)
            start_gather(c1, rows_b)

            @pl.when(j > 0)
            def _():
                wait_out(out_a)
                wait_out(out_b)

            compact(c0, rows_a, out_a)
            emit_out(c0, out_a)

            wait_gather(rows_b)

            @pl.when(c1 + 1 < n_chunks)
            def _():
                start_gather(c1 + 1, rows_a)

            compact(c1, rows_b, out_b)
            emit_out(c1, out_b)
            return carry

        lax.fori_loop(0, n_chunks // 2, body, 0)
        wait_out(out_a)
        wait_out(out_b)

    return emb_kernel


def kernel(x, weight):
    b, s = x.shape
    n_rows = b * s
    xf = x.reshape(n_rows).astype(jnp.int32)
    p2d = lax.shift_right_logical(xf, 1).reshape(n_rows // CHUNK, CHUNK)
    h2d = lax.shift_left(jnp.bitwise_and(xf, 1), 6).reshape(
        n_rows // CHUNK, CHUNK)
    wp = weight.reshape(weight.shape[0] // 2, 2 * EMBED)
    out = _make_kernel(n_rows)(p2d, h2d, wp)
    return out.reshape(b, s, EMBED)


# speedup vs baseline: 1.6818x; 1.5274x over previous
import functools

import jax
import jax.numpy as jnp
from jax import lax
from jax.experimental import pallas as pl
from jax.experimental.pallas import tpu as pltpu
from jax.experimental.pallas import tpu_sc as plsc

EMBED = 64
NC = 2
NS = 16
NW = NC * NS
CHUNK = 128
LANES = 16
GROUPS = CHUNK // LANES


@functools.lru_cache(maxsize=None)
def _make_kernel(n_rows: int):
    n_per_w = n_rows // NW
    n_chunks = n_per_w // CHUNK

    mesh = plsc.VectorSubcoreMesh(core_axis_name="c", subcore_axis_name="s")

    @functools.partial(
        pl.kernel,
        mesh=mesh,
        out_type=jax.ShapeDtypeStruct((n_rows, EMBED), jnp.float32),
        scratch_types=[
            pltpu.VMEM((n_chunks, CHUNK), jnp.int32),
            pltpu.VMEM((n_chunks, CHUNK), jnp.int32),
            pltpu.VMEM((CHUNK, 2 * EMBED), jnp.float32),
            pltpu.VMEM((CHUNK, 2 * EMBED), jnp.float32),
            pltpu.VMEM((CHUNK, EMBED), jnp.float32),
            pltpu.VMEM((CHUNK, EMBED), jnp.float32),
            pltpu.SemaphoreType.DMA,
            pltpu.SemaphoreType.DMA,
        ],
        compiler_params=pltpu.CompilerParams(needs_layout_passes=False),
    )
    def emb_kernel(p_hbm, h_hbm, wp_hbm, out_hbm,
                   pv, hv, rows_a, rows_b, out_a, out_b, gsem, osem):
        wid = lax.axis_index("s") * NC + lax.axis_index("c")
        cbase = wid * n_chunks
        base = wid * n_per_w
        iota = lax.iota(jnp.int32, LANES)

        pltpu.sync_copy(p_hbm.at[pl.ds(cbase, n_chunks)], pv)
        pltpu.sync_copy(h_hbm.at[pl.ds(cbase, n_chunks)], hv)

        def start_gather(c, rbuf):
            return pltpu.async_copy(wp_hbm.at[pv.at[c]], rbuf, gsem)

        def wait_gather(rbuf):
            pltpu.make_async_copy(wp_hbm.at[pv.at[0]], rbuf, gsem).wait()

        def wait_out(obuf):
            pltpu.make_async_copy(obuf, out_hbm.at[pl.ds(base, CHUNK)],
                                  osem).wait()

        def compact(c, rbuf, obuf):
            for g in range(GROUPS):
                hvec = hv[c, pl.ds(g * LANES, LANES)]
                for k in range(LANES):
                    i = g * LANES + k
                    h64 = hvec[k]
                    for q in range(EMBED // LANES):
                        obuf[i, pl.ds(q * LANES, LANES)] = (
                            rbuf[i, pl.ds(h64 + q * LANES, LANES)]
                        )

        def emit_out(c, obuf):
            pltpu.async_copy(obuf, out_hbm.at[pl.ds(base + c * CHUNK, CHUNK)],
                             osem)

        start_gather(0, rows_a)

        def body(j, carry):
            c0 = 2 * j
            c1 = c0 + 1

            wait_gather(rows_a)
            start_gather(c1, rows_b)

            @pl.when(j > 0)
            def _():
                wait_out(out_a)
                wait_out(out_b)

            compact(c0, rows_a, out_a)
            emit_out(c0, out_a)

            wait_gather(rows_b)

            @pl.when(c1 + 1 < n_chunks)
            def _():
                start_gather(c1 + 1, rows_a)

            compact(c1, rows_b, out_b)
            emit_out(c1, out_b)
            return carry

        lax.fori_loop(0, n_chunks // 2, body, 0)
        wait_out(out_a)
        wait_out(out_b)

    return emb_kernel


def kernel(x, weight):
    b, s = x.shape
    n_rows = b * s
    xf = x.reshape(n_rows).astype(jnp.int32)
    p2d = lax.shift_right_logical(xf, 1).reshape(n_rows // CHUNK, CHUNK)
    h2d = lax.shift_left(jnp.bitwise_and(xf, 1), 6).reshape(
        n_rows // CHUNK, CHUNK)
    wp = weight.reshape(weight.shape[0] // 2, 2 * EMBED)
    out = _make_kernel(n_rows)(p2d, h2d, wp)
    return out.reshape(b, s, EMBED)
